# baseline (speedup 1.0000x reference)
import functools

import jax
import jax.numpy as jnp
from jax import lax
from jax.experimental import pallas as pl
from jax.experimental.pallas import tpu as pltpu
from jax.experimental.pallas import tpu_sc as plsc

VOCAB = 1000000
D = 64
BATCH = 4096
SEQ = 200

NUM_CORES = 2
NUM_SUBCORES = 16
NW = NUM_CORES * NUM_SUBCORES

NFULL = VOCAB // 128
TAIL = VOCAB - NFULL * 128
NBUF = 4
LEAD = 2

_B_PER_W = BATCH * SEQ // NW
_NBLK = _B_PER_W // 128


def _mesh():
    return plsc.VectorSubcoreMesh(core_axis_name="c", subcore_axis_name="s")


def _iota16():
    return lax.iota(jnp.int32, 16)


@functools.lru_cache(maxsize=None)
def _repack_kernel():
    @functools.partial(
        pl.kernel,
        mesh=_mesh(),
        out_type=jax.ShapeDtypeStruct((VOCAB // 2, 2 * D), jnp.float32),
        compiler_params=pltpu.CompilerParams(
            use_tc_tiling_on_sc=True, needs_layout_passes=False
        ),
        scratch_types=[
            pltpu.VMEM((NBUF, D, 128), jnp.float32),
            pltpu.VMEM((NBUF, D, 128), jnp.float32),
            pltpu.SemaphoreType.DMA((NBUF,)),
            pltpu.SemaphoreType.DMA((NBUF,)),
        ],
    )
    def repack(lutT_hbm, out_hbm, tbuf, obuf, isem, osem):
        wid = lax.axis_index("s") * NUM_CORES + lax.axis_index("c")
        cnt = jnp.where(wid < NFULL % NW, NFULL // NW + 1, NFULL // NW)

        def v0_of(t):
            return pl.multiple_of(128 * (wid + NW * t), 128)

        def start_in(t, b):
            pltpu.make_async_copy(
                lutT_hbm.at[:, pl.ds(v0_of(t), 128)], tbuf.at[b], isem.at[b]
            ).start()

        def wait_in(b):
            pltpu.make_async_copy(
                lutT_hbm.at[:, pl.ds(0, 128)], tbuf.at[b], isem.at[b]
            ).wait()

        def start_out(t, b):
            pltpu.make_async_copy(
                obuf.at[b],
                out_hbm.at[pl.ds(pl.multiple_of(64 * (wid + NW * t), 64), 64)],
                osem.at[b],
            ).start()

        def wait_out(b):
            pltpu.make_async_copy(
                obuf.at[b], out_hbm.at[pl.ds(0, 64)], osem.at[b]
            ).wait()

        def transpose_block(b, ncols):
            def col(c, carry):
                cs = jnp.full((16,), c, jnp.int32)
                for k in range(4):
                    vec = plsc.load_gather(tbuf.at[b], [_iota16() + 16 * k, cs])
                    obuf[b, c >> 1, pl.ds((c & 1) * 64 + 16 * k, 16)] = vec
                return carry

            lax.fori_loop(0, ncols, col, 0)

        def step(t, b):
            bg = (b + LEAD) % NBUF

            @pl.when(t + LEAD < cnt)
            def _():
                start_in(t + LEAD, bg)

            wait_in(b)

            @pl.when(t >= NBUF)
            def _():
                wait_out(b)

            transpose_block(b, 128)
            start_out(t, b)

        for b in range(LEAD):
            start_in(b, b)

        def group(g, carry):
            for b in range(NBUF):
                step(g * NBUF + b, b)
            return carry

        lax.fori_loop(0, (NFULL // NW) // NBUF, group, 0)

        @pl.when(wid < NFULL % NW)
        def _():
            step(NFULL // NW, 0)

        for b in range(NBUF):
            wait_out(b)

    return repack


@functools.lru_cache(maxsize=None)
def _gather_kernel():
    @functools.partial(
        pl.kernel,
        mesh=_mesh(),
        out_type=jax.ShapeDtypeStruct((SEQ, 8, 32, 8, 128), jnp.float32),
        compiler_params=pltpu.CompilerParams(
            use_tc_tiling_on_sc=False, needs_layout_passes=False
        ),
        scratch_types=[
            pltpu.VMEM((_NBLK, 128), jnp.int32),
            pltpu.VMEM((NBUF, 128, D), jnp.float32),
            pltpu.VMEM((NBUF, 8, 8, 128), jnp.float32),
            pltpu.VMEM((TAIL, D), jnp.float32),
            pltpu.SemaphoreType.DMA((NBUF,)),
            pltpu.SemaphoreType.DMA((NBUF,)),
        ],
    )
    def gather(xT_hbm, lut_hbm, tail_hbm, out_hbm, idx_v, rows, blk, tailv,
               gsem, osem):
        wid = lax.axis_index("s") * NUM_CORES + lax.axis_index("c")
        pltpu.sync_copy(xT_hbm.at[wid], idx_v)
        pltpu.sync_copy(tail_hbm, tailv)

        def start_gather(t, b):
            pltpu.make_async_copy(
                lut_hbm.at[idx_v.at[t]], rows.at[b], gsem.at[b]
            ).start()

        def wait_gather(b):
            pltpu.make_async_copy(
                lut_hbm.at[idx_v.at[0]], rows.at[b], gsem.at[b]
            ).wait()

        def out_slice(t):
            T = wid * _NBLK + t
            return out_hbm.at[T >> 5, :, T & 31]

        def start_out(t, b):
            pltpu.make_async_copy(blk.at[b], out_slice(t), osem.at[b]).start()

        def wait_out(b):
            pltpu.make_async_copy(blk.at[b], out_slice(0), osem.at[b]).wait()

        def transpose_block(b):
            def row(d, carry):
                ds_ = jnp.full((16,), d, jnp.int32)
                for g in range(8):
                    vec = plsc.load_gather(rows.at[b], [_iota16() + 16 * g, ds_])
                    blk[b, d >> 3, d & 7, pl.ds(16 * g, 16)] = vec
                return carry

            lax.fori_loop(0, D, row, 0)

        def fix_tail(t, b):
            n_tail = jnp.int32(0)
            for g in range(8):
                iv = idx_v[t, pl.ds(16 * g, 16)]
                n_tail = n_tail + jnp.sum((iv >= NFULL * 128).astype(jnp.int32))

            @pl.when(n_tail > 0)
            def _():
                def fix(d, carry):
                    dsp = jnp.full((16,), d, jnp.int32)
                    for g in range(8):
                        iv = idx_v[t, pl.ds(16 * g, 16)]
                        m = iv >= NFULL * 128
                        tidx = jnp.maximum(iv - NFULL * 128, 0)
                        vals = plsc.load_gather(tailv, [tidx, dsp], mask=m)
                        plsc.store_scatter(
                            rows.at[b], [_iota16() + 16 * g, dsp], vals, mask=m
                        )
                    return carry

                lax.fori_loop(0, D, fix, 0)

        def step(t, b):
            bg = (b + LEAD) % NBUF

            @pl.when(t + LEAD < _NBLK)
            def _():
                start_gather(t + LEAD, bg)

            wait_gather(b)
            fix_tail(t, b)

            @pl.when(t >= NBUF)
            def _():
                wait_out(b)

            transpose_block(b)
            start_out(t, b)

        for b in range(LEAD):
            start_gather(b, b)

        def group(g, carry):
            for b in range(NBUF):
                step(g * NBUF + b, b)
            return carry

        lax.fori_loop(0, _NBLK // NBUF, group, 0)

        for b in range(NBUF):
            wait_out(b)

    return gather


def kernel(x, lut):
    lutT = jnp.transpose(lut)
    lut_pk = _repack_kernel()(lutT)
    lut_rm = lut_pk.reshape(VOCAB, D)
    xT3 = jnp.transpose(x).astype(jnp.int32).reshape(NW, _NBLK, 128)
    lut_tail = lut[NFULL * 128:, :]
    out5 = _gather_kernel()(xT3, lut_rm, lut_tail)
    return out5.transpose(2, 4, 0, 1, 3).reshape(BATCH, SEQ, D)


# speedup vs baseline: 1.8016x; 1.8016x over previous
import functools

import jax
import jax.numpy as jnp
from jax import lax
from jax.experimental import pallas as pl
from jax.experimental.pallas import tpu as pltpu
from jax.experimental.pallas import tpu_sc as plsc

VOCAB = 1000000
D = 64
BATCH = 4096
SEQ = 200

NUM_CORES = 2
NUM_SUBCORES = 16
NW = NUM_CORES * NUM_SUBCORES

NFULL = VOCAB // 128
TAIL = VOCAB - NFULL * 128
NBUF = 4
LEAD = 2

_B_PER_W = BATCH * SEQ // NW
_NBLK = _B_PER_W // 128


def _mesh():
    return plsc.VectorSubcoreMesh(core_axis_name="c", subcore_axis_name="s")


def _iota16():
    return lax.iota(jnp.int32, 16)


@functools.lru_cache(maxsize=None)
def _repack_kernel():
    @functools.partial(
        pl.kernel,
        mesh=_mesh(),
        out_type=jax.ShapeDtypeStruct((VOCAB // 2, 2 * D), jnp.float32),
        compiler_params=pltpu.CompilerParams(
            use_tc_tiling_on_sc=True, needs_layout_passes=False
        ),
        scratch_types=[
            pltpu.VMEM((NBUF, D, 128), jnp.float32),
            pltpu.VMEM((NBUF, D, 128), jnp.float32),
            pltpu.SemaphoreType.DMA((NBUF,)),
            pltpu.SemaphoreType.DMA((NBUF,)),
        ],
    )
    def repack(lutT_hbm, out_hbm, tbuf, obuf, isem, osem):
        wid = lax.axis_index("s") * NUM_CORES + lax.axis_index("c")
        cnt = jnp.where(wid < NFULL % NW, NFULL // NW + 1, NFULL // NW)

        def v0_of(t):
            return pl.multiple_of(128 * (wid + NW * t), 128)

        def start_in(t, b):
            pltpu.make_async_copy(
                lutT_hbm.at[:, pl.ds(v0_of(t), 128)], tbuf.at[b], isem.at[b]
            ).start()

        def wait_in(b):
            pltpu.make_async_copy(
                lutT_hbm.at[:, pl.ds(0, 128)], tbuf.at[b], isem.at[b]
            ).wait()

        def start_out(t, b):
            pltpu.make_async_copy(
                obuf.at[b],
                out_hbm.at[pl.ds(pl.multiple_of(64 * (wid + NW * t), 64), 64)],
                osem.at[b],
            ).start()

        def wait_out(b):
            pltpu.make_async_copy(
                obuf.at[b], out_hbm.at[pl.ds(0, 64)], osem.at[b]
            ).wait()

        dvecs = [_iota16() + 16 * k for k in range(4)]

        def transpose_block(b, ncols):
            @plsc.parallel_loop(0, ncols, unroll=8)
            def _(c):
                cs = jnp.full((16,), c, jnp.int32)
                for k in range(4):
                    vec = plsc.load_gather(tbuf.at[b], [dvecs[k], cs])
                    obuf[b, c >> 1, pl.ds((c & 1) * 64 + 16 * k, 16)] = vec

        def step(t, b):
            bg = (b + LEAD) % NBUF

            @pl.when(t + LEAD < cnt)
            def _():
                start_in(t + LEAD, bg)

            wait_in(b)

            @pl.when(t >= NBUF)
            def _():
                wait_out(b)

            transpose_block(b, 128)
            start_out(t, b)

        for b in range(LEAD):
            start_in(b, b)

        def group(g, carry):
            for b in range(NBUF):
                step(g * NBUF + b, b)
            return carry

        lax.fori_loop(0, (NFULL // NW) // NBUF, group, 0)

        @pl.when(wid < NFULL % NW)
        def _():
            step(NFULL // NW, 0)

        for b in range(NBUF):
            wait_out(b)

    return repack


@functools.lru_cache(maxsize=None)
def _gather_kernel():
    @functools.partial(
        pl.kernel,
        mesh=_mesh(),
        out_type=jax.ShapeDtypeStruct((SEQ, 8, 32, 8, 128), jnp.float32),
        compiler_params=pltpu.CompilerParams(
            use_tc_tiling_on_sc=False, needs_layout_passes=False
        ),
        scratch_types=[
            pltpu.VMEM((_NBLK, 128), jnp.int32),
            pltpu.VMEM((NBUF, 128, D), jnp.float32),
            pltpu.VMEM((NBUF, 8, 8, 128), jnp.float32),
            pltpu.VMEM((TAIL, D), jnp.float32),
            pltpu.SemaphoreType.DMA((NBUF,)),
            pltpu.SemaphoreType.DMA((NBUF,)),
        ],
    )
    def gather(xT_hbm, lut_hbm, tail_hbm, out_hbm, idx_v, rows, blk, tailv,
               gsem, osem):
        wid = lax.axis_index("s") * NUM_CORES + lax.axis_index("c")
        pltpu.sync_copy(xT_hbm.at[wid], idx_v)
        pltpu.sync_copy(tail_hbm, tailv)

        def start_gather(t, b):
            pltpu.make_async_copy(
                lut_hbm.at[idx_v.at[t]], rows.at[b], gsem.at[b]
            ).start()

        def wait_gather(b):
            pltpu.make_async_copy(
                lut_hbm.at[idx_v.at[0]], rows.at[b], gsem.at[b]
            ).wait()

        def out_slice(t):
            T = wid * _NBLK + t
            return out_hbm.at[T >> 5, :, T & 31]

        def start_out(t, b):
            pltpu.make_async_copy(blk.at[b], out_slice(t), osem.at[b]).start()

        def wait_out(b):
            pltpu.make_async_copy(blk.at[b], out_slice(0), osem.at[b]).wait()

        cvecs = [_iota16() + 16 * g for g in range(8)]

        def transpose_block(b):
            @plsc.parallel_loop(0, D, unroll=8)
            def _(d):
                ds_ = jnp.full((16,), d, jnp.int32)
                for g in range(8):
                    vec = plsc.load_gather(rows.at[b], [cvecs[g], ds_])
                    blk[b, d >> 3, d & 7, pl.ds(16 * g, 16)] = vec

        def fix_tail(t, b):
            n_tail = jnp.int32(0)
            for g in range(8):
                iv = idx_v[t, pl.ds(16 * g, 16)]
                n_tail = n_tail + jnp.sum((iv >= NFULL * 128).astype(jnp.int32))

            @pl.when(n_tail > 0)
            def _():
                def fix(d, carry):
                    dsp = jnp.full((16,), d, jnp.int32)
                    for g in range(8):
                        iv = idx_v[t, pl.ds(16 * g, 16)]
                        m = iv >= NFULL * 128
                        tidx = jnp.maximum(iv - NFULL * 128, 0)
                        vals = plsc.load_gather(tailv, [tidx, dsp], mask=m)
                        plsc.store_scatter(
                            rows.at[b], [_iota16() + 16 * g, dsp], vals, mask=m
                        )
                    return carry

                lax.fori_loop(0, D, fix, 0)

        def step(t, b):
            bg = (b + LEAD) % NBUF

            @pl.when(t + LEAD < _NBLK)
            def _():
                start_gather(t + LEAD, bg)

            wait_gather(b)
            fix_tail(t, b)

            @pl.when(t >= NBUF)
            def _():
                wait_out(b)

            transpose_block(b)
            start_out(t, b)

        for b in range(LEAD):
            start_gather(b, b)

        def group(g, carry):
            for b in range(NBUF):
                step(g * NBUF + b, b)
            return carry

        lax.fori_loop(0, _NBLK // NBUF, group, 0)

        for b in range(NBUF):
            wait_out(b)

    return gather


def kernel(x, lut):
    lutT = jnp.transpose(lut)
    lut_pk = _repack_kernel()(lutT)
    lut_rm = lut_pk.reshape(VOCAB, D)
    xT3 = jnp.transpose(x).astype(jnp.int32).reshape(NW, _NBLK, 128)
    lut_tail = lut[NFULL * 128:, :]
    out5 = _gather_kernel()(xT3, lut_rm, lut_tail)
    return out5.transpose(2, 4, 0, 1, 3).reshape(BATCH, SEQ, D)


# speedup vs baseline: 1.8224x; 1.0116x over previous
import functools

import jax
import jax.numpy as jnp
from jax import lax
from jax.experimental import pallas as pl
from jax.experimental.pallas import tpu as pltpu
from jax.experimental.pallas import tpu_sc as plsc

VOCAB = 1000000
D = 64
BATCH = 4096
SEQ = 200

NUM_CORES = 2
NUM_SUBCORES = 16
NW = NUM_CORES * NUM_SUBCORES

NFULL = VOCAB // 128
TAIL = VOCAB - NFULL * 128
NBUF = 4
LEAD = 2

_B_PER_W = BATCH * SEQ // NW
_NBLK = _B_PER_W // 128


def _mesh():
    return plsc.VectorSubcoreMesh(core_axis_name="c", subcore_axis_name="s")


def _iota16():
    return lax.iota(jnp.int32, 16)


@functools.lru_cache(maxsize=None)
def _repack_kernel():
    @functools.partial(
        pl.kernel,
        mesh=_mesh(),
        out_type=jax.ShapeDtypeStruct((VOCAB // 2, 2 * D), jnp.float32),
        compiler_params=pltpu.CompilerParams(
            use_tc_tiling_on_sc=True, needs_layout_passes=False
        ),
        scratch_types=[
            pltpu.VMEM((NBUF, D, 128), jnp.float32),
            pltpu.VMEM((NBUF, D, 128), jnp.float32),
            pltpu.SemaphoreType.DMA((NBUF,)),
            pltpu.SemaphoreType.DMA((NBUF,)),
        ],
    )
    def repack(lutT_hbm, out_hbm, tbuf, obuf, isem, osem):
        wid = lax.axis_index("s") * NUM_CORES + lax.axis_index("c")
        cnt = jnp.where(wid < NFULL % NW, NFULL // NW + 1, NFULL // NW)

        def v0_of(t):
            return pl.multiple_of(128 * (wid + NW * t), 128)

        def start_in(t, b):
            pltpu.make_async_copy(
                lutT_hbm.at[:, pl.ds(v0_of(t), 128)], tbuf.at[b], isem.at[b]
            ).start()

        def wait_in(b):
            pltpu.make_async_copy(
                lutT_hbm.at[:, pl.ds(0, 128)], tbuf.at[b], isem.at[b]
            ).wait()

        def start_out(t, b):
            pltpu.make_async_copy(
                obuf.at[b],
                out_hbm.at[pl.ds(pl.multiple_of(64 * (wid + NW * t), 64), 64)],
                osem.at[b],
            ).start()

        def wait_out(b):
            pltpu.make_async_copy(
                obuf.at[b], out_hbm.at[pl.ds(0, 64)], osem.at[b]
            ).wait()

        dvecs = [_iota16() + 16 * k for k in range(4)]

        def transpose_block(b, ncols):
            @plsc.parallel_loop(0, ncols, unroll=16)
            def _(c):
                cs = jnp.full((16,), c, jnp.int32)
                for k in range(4):
                    vec = plsc.load_gather(tbuf.at[b], [dvecs[k], cs])
                    obuf[b, c >> 1, pl.ds((c & 1) * 64 + 16 * k, 16)] = vec

        def step(t, b):
            bg = (b + LEAD) % NBUF

            @pl.when(t + LEAD < cnt)
            def _():
                start_in(t + LEAD, bg)

            wait_in(b)

            @pl.when(t >= NBUF)
            def _():
                wait_out(b)

            transpose_block(b, 128)
            start_out(t, b)

        for b in range(LEAD):
            start_in(b, b)

        def group(g, carry):
            for b in range(NBUF):
                step(g * NBUF + b, b)
            return carry

        lax.fori_loop(0, (NFULL // NW) // NBUF, group, 0)

        @pl.when(wid < NFULL % NW)
        def _():
            step(NFULL // NW, 0)

        for b in range(NBUF):
            wait_out(b)

    return repack


@functools.lru_cache(maxsize=None)
def _gather_kernel():
    @functools.partial(
        pl.kernel,
        mesh=_mesh(),
        out_type=jax.ShapeDtypeStruct((SEQ, 8, 32, 8, 128), jnp.float32),
        compiler_params=pltpu.CompilerParams(
            use_tc_tiling_on_sc=False, needs_layout_passes=False
        ),
        scratch_types=[
            pltpu.VMEM((_NBLK, 128), jnp.int32),
            pltpu.VMEM((NBUF, 128, D), jnp.float32),
            pltpu.VMEM((NBUF, 8, 8, 128), jnp.float32),
            pltpu.VMEM((TAIL, D), jnp.float32),
            pltpu.SemaphoreType.DMA((NBUF,)),
            pltpu.SemaphoreType.DMA((NBUF,)),
        ],
    )
    def gather(xT_hbm, lut_hbm, tail_hbm, out_hbm, idx_v, rows, blk, tailv,
               gsem, osem):
        wid = lax.axis_index("s") * NUM_CORES + lax.axis_index("c")
        pltpu.sync_copy(xT_hbm.at[wid], idx_v)
        pltpu.sync_copy(tail_hbm, tailv)

        def start_gather(t, b):
            pltpu.make_async_copy(
                lut_hbm.at[idx_v.at[t]], rows.at[b], gsem.at[b]
            ).start()

        def wait_gather(b):
            pltpu.make_async_copy(
                lut_hbm.at[idx_v.at[0]], rows.at[b], gsem.at[b]
            ).wait()

        def out_slice(t):
            T = wid * _NBLK + t
            return out_hbm.at[T >> 5, :, T & 31]

        def start_out(t, b):
            pltpu.make_async_copy(blk.at[b], out_slice(t), osem.at[b]).start()

        def wait_out(b):
            pltpu.make_async_copy(blk.at[b], out_slice(0), osem.at[b]).wait()

        cvecs = [_iota16() + 16 * g for g in range(8)]

        def transpose_block(b):
            @plsc.parallel_loop(0, D, unroll=16)
            def _(d):
                ds_ = jnp.full((16,), d, jnp.int32)
                for g in range(8):
                    vec = plsc.load_gather(rows.at[b], [cvecs[g], ds_])
                    blk[b, d >> 3, d & 7, pl.ds(16 * g, 16)] = vec

        def fix_tail(t, b):
            n_tail = jnp.int32(0)
            for g in range(8):
                iv = idx_v[t, pl.ds(16 * g, 16)]
                n_tail = n_tail + jnp.sum((iv >= NFULL * 128).astype(jnp.int32))

            @pl.when(n_tail > 0)
            def _():
                def fix(d, carry):
                    dsp = jnp.full((16,), d, jnp.int32)
                    for g in range(8):
                        iv = idx_v[t, pl.ds(16 * g, 16)]
                        m = iv >= NFULL * 128
                        tidx = jnp.maximum(iv - NFULL * 128, 0)
                        vals = plsc.load_gather(tailv, [tidx, dsp], mask=m)
                        plsc.store_scatter(
                            rows.at[b], [_iota16() + 16 * g, dsp], vals, mask=m
                        )
                    return carry

                lax.fori_loop(0, D, fix, 0)

        def step(t, b):
            bg = (b + LEAD) % NBUF

            @pl.when(t + LEAD < _NBLK)
            def _():
                start_gather(t + LEAD, bg)

            wait_gather(b)
            fix_tail(t, b)

            @pl.when(t >= NBUF)
            def _():
                wait_out(b)

            transpose_block(b)
            start_out(t, b)

        for b in range(LEAD):
            start_gather(b, b)

        def group(g, carry):
            for b in range(NBUF):
                step(g * NBUF + b, b)
            return carry

        lax.fori_loop(0, _NBLK // NBUF, group, 0)

        for b in range(NBUF):
            wait_out(b)

    return gather


def kernel(x, lut):
    lutT = jnp.transpose(lut)
    lut_pk = _repack_kernel()(lutT)
    lut_rm = lut_pk.reshape(VOCAB, D)
    xT3 = jnp.transpose(x).astype(jnp.int32).reshape(NW, _NBLK, 128)
    lut_tail = lut[NFULL * 128:, :]
    out5 = _gather_kernel()(xT3, lut_rm, lut_tail)
    return out5.transpose(2, 4, 0, 1, 3).reshape(BATCH, SEQ, D)


# speedup vs baseline: 3.6639x; 2.0105x over previous
import functools

import jax
import jax.numpy as jnp
from jax import lax
from jax.experimental import pallas as pl
from jax.experimental.pallas import tpu as pltpu
from jax.experimental.pallas import tpu_sc as plsc

VOCAB = 1000000
D = 64
BATCH = 4096
SEQ = 200

NUM_CORES = 2
NUM_SUBCORES = 16
NW = NUM_CORES * NUM_SUBCORES

TBLK = 8192
NBUF = 5
LEAD = 2

_B_PER_W = BATCH * SEQ // NW
_NBLK = _B_PER_W // 128


def _mesh():
    return plsc.VectorSubcoreMesh(core_axis_name="c", subcore_axis_name="s")


def _repack_tc(lutT):
    def body(x_ref, o_ref):
        o_ref[:, 0:D] = jnp.transpose(x_ref[...])

    grid = (VOCAB + TBLK - 1) // TBLK
    return pl.pallas_call(
        body,
        grid=(grid,),
        in_specs=[pl.BlockSpec((D, TBLK), lambda i: (0, i))],
        out_specs=pl.BlockSpec((TBLK, 2 * D), lambda i: (i, 0)),
        out_shape=jax.ShapeDtypeStruct((VOCAB, 2 * D), jnp.float32),
    )(lutT)


@functools.lru_cache(maxsize=None)
def _gather_kernel():
    @functools.partial(
        pl.kernel,
        mesh=_mesh(),
        out_type=jax.ShapeDtypeStruct((BATCH * SEQ, 2 * D), jnp.float32),
        compiler_params=pltpu.CompilerParams(
            use_tc_tiling_on_sc=False, needs_layout_passes=False
        ),
        scratch_types=[
            pltpu.VMEM((_NBLK, 128), jnp.int32),
            pltpu.VMEM((NBUF, 128, 2 * D), jnp.float32),
            pltpu.SemaphoreType.DMA((NBUF,)),
            pltpu.SemaphoreType.DMA((NBUF,)),
        ],
    )
    def gather(x_hbm, lut_hbm, out_hbm, idx_v, rows, gsem, osem):
        wid = lax.axis_index("s") * NUM_CORES + lax.axis_index("c")
        base = wid * _B_PER_W
        pltpu.sync_copy(x_hbm.at[wid], idx_v)

        def start_gather(t, b):
            pltpu.make_async_copy(
                lut_hbm.at[idx_v.at[t]], rows.at[b], gsem.at[b]
            ).start()

        def wait_gather(b):
            pltpu.make_async_copy(
                lut_hbm.at[idx_v.at[0]], rows.at[b], gsem.at[b]
            ).wait()

        def start_out(t, b):
            pltpu.make_async_copy(
                rows.at[b], out_hbm.at[pl.ds(base + 128 * t, 128)], osem.at[b]
            ).start()

        def wait_out(b):
            pltpu.make_async_copy(
                rows.at[b], out_hbm.at[pl.ds(base, 128)], osem.at[b]
            ).wait()

        def step(t, b):
            bg = (b + LEAD) % NBUF

            @pl.when(t + LEAD < _NBLK)
            def _():
                wait_out(bg)
                start_gather(t + LEAD, bg)

            wait_gather(b)
            start_out(t, b)

        for b in range(LEAD):
            start_gather(b, b)

        for b in range(NBUF):
            bg = (b + LEAD) % NBUF
            if b >= NBUF - LEAD:
                wait_out(bg)
            start_gather(b + LEAD, bg)
            wait_gather(b)
            start_out(b, b)

        def group(g, carry):
            for b in range(NBUF):
                step(g * NBUF + b, b)
            return carry

        lax.fori_loop(1, _NBLK // NBUF, group, 0)

        for b in range(NBUF):
            wait_out(b)

    return gather


def kernel(x, lut):
    lutT = jnp.transpose(lut)
    lut_pk = _repack_tc(lutT)
    x3 = x.astype(jnp.int32).reshape(NW, _NBLK, 128)
    out = _gather_kernel()(x3, lut_pk)
    return out[:, :D].reshape(BATCH, SEQ, D)


# speedup vs baseline: 3.7666x; 1.0280x over previous
import functools

import jax
import jax.numpy as jnp
from jax import lax
from jax.experimental import pallas as pl
from jax.experimental.pallas import tpu as pltpu
from jax.experimental.pallas import tpu_sc as plsc

VOCAB = 1000000
D = 64
BATCH = 4096
SEQ = 200

NUM_CORES = 2
NUM_SUBCORES = 16
NW = NUM_CORES * NUM_SUBCORES

TBLK = 16384
NBUF = 5
LEAD = 2

_B_PER_W = BATCH * SEQ // NW
_NBLK = _B_PER_W // 128


def _mesh():
    return plsc.VectorSubcoreMesh(core_axis_name="c", subcore_axis_name="s")


def _repack_tc(lutT):
    def body(x_ref, o_ref):
        o_ref[:, 0:D] = jnp.transpose(x_ref[...])

    grid = (VOCAB + TBLK - 1) // TBLK
    return pl.pallas_call(
        body,
        grid=(grid,),
        in_specs=[pl.BlockSpec((D, TBLK), lambda i: (0, i))],
        out_specs=pl.BlockSpec((TBLK, 2 * D), lambda i: (i, 0)),
        out_shape=jax.ShapeDtypeStruct((VOCAB, 2 * D), jnp.float32),
    )(lutT)


@functools.lru_cache(maxsize=None)
def _gather_kernel():
    @functools.partial(
        pl.kernel,
        mesh=_mesh(),
        out_type=jax.ShapeDtypeStruct((BATCH * SEQ, 2 * D), jnp.float32),
        compiler_params=pltpu.CompilerParams(
            use_tc_tiling_on_sc=False, needs_layout_passes=False
        ),
        scratch_types=[
            pltpu.VMEM((_NBLK, 128), jnp.int32),
            pltpu.VMEM((NBUF, 128, 2 * D), jnp.float32),
            pltpu.SemaphoreType.DMA((NBUF,)),
            pltpu.SemaphoreType.DMA((NBUF,)),
        ],
    )
    def gather(x_hbm, lut_hbm, out_hbm, idx_v, rows, gsem, osem):
        wid = lax.axis_index("s") * NUM_CORES + lax.axis_index("c")
        base = wid * _B_PER_W
        pltpu.sync_copy(x_hbm.at[wid], idx_v)

        def start_gather(t, b):
            pltpu.make_async_copy(
                lut_hbm.at[idx_v.at[t]], rows.at[b], gsem.at[b]
            ).start()

        def wait_gather(b):
            pltpu.make_async_copy(
                lut_hbm.at[idx_v.at[0]], rows.at[b], gsem.at[b]
            ).wait()

        def start_out(t, b):
            pltpu.make_async_copy(
                rows.at[b], out_hbm.at[pl.ds(base + 128 * t, 128)], osem.at[b]
            ).start()

        def wait_out(b):
            pltpu.make_async_copy(
                rows.at[b], out_hbm.at[pl.ds(base, 128)], osem.at[b]
            ).wait()

        def step(t, b):
            bg = (b + LEAD) % NBUF

            @pl.when(t + LEAD < _NBLK)
            def _():
                wait_out(bg)
                start_gather(t + LEAD, bg)

            wait_gather(b)
            start_out(t, b)

        for b in range(LEAD):
            start_gather(b, b)

        for b in range(NBUF):
            bg = (b + LEAD) % NBUF
            if b >= NBUF - LEAD:
                wait_out(bg)
            start_gather(b + LEAD, bg)
            wait_gather(b)
            start_out(b, b)

        def group(g, carry):
            for b in range(NBUF):
                step(g * NBUF + b, b)
            return carry

        lax.fori_loop(1, _NBLK // NBUF, group, 0)

        for b in range(NBUF):
            wait_out(b)

    return gather


def kernel(x, lut):
    lutT = jnp.transpose(lut)
    lut_pk = _repack_tc(lutT)
    x3 = x.astype(jnp.int32).reshape(NW, _NBLK, 128)
    out = _gather_kernel()(x3, lut_pk)
    return out[:, :D].reshape(BATCH, SEQ, D)


# speedup vs baseline: 3.8121x; 1.0121x over previous
import functools

import jax
import jax.numpy as jnp
from jax import lax
from jax.experimental import pallas as pl
from jax.experimental.pallas import tpu as pltpu
from jax.experimental.pallas import tpu_sc as plsc

VOCAB = 1000000
D = 64
BATCH = 4096
SEQ = 200

NUM_CORES = 2
NUM_SUBCORES = 16
NW = NUM_CORES * NUM_SUBCORES

TBLK = 32768
NBUF = 5
LEAD = 2

_B_PER_W = BATCH * SEQ // NW
_NBLK = _B_PER_W // 128


def _mesh():
    return plsc.VectorSubcoreMesh(core_axis_name="c", subcore_axis_name="s")


def _repack_tc(lutT):
    def body(x_ref, o_ref):
        o_ref[:, 0:D] = jnp.transpose(x_ref[...])

    grid = (VOCAB + TBLK - 1) // TBLK
    return pl.pallas_call(
        body,
        grid=(grid,),
        in_specs=[pl.BlockSpec((D, TBLK), lambda i: (0, i))],
        out_specs=pl.BlockSpec((TBLK, 2 * D), lambda i: (i, 0)),
        out_shape=jax.ShapeDtypeStruct((VOCAB, 2 * D), jnp.float32),
    )(lutT)


@functools.lru_cache(maxsize=None)
def _gather_kernel():
    @functools.partial(
        pl.kernel,
        mesh=_mesh(),
        out_type=jax.ShapeDtypeStruct((BATCH * SEQ, 2 * D), jnp.float32),
        compiler_params=pltpu.CompilerParams(
            use_tc_tiling_on_sc=False, needs_layout_passes=False
        ),
        scratch_types=[
            pltpu.VMEM((_NBLK, 128), jnp.int32),
            pltpu.VMEM((NBUF, 128, 2 * D), jnp.float32),
            pltpu.SemaphoreType.DMA((NBUF,)),
            pltpu.SemaphoreType.DMA((NBUF,)),
        ],
    )
    def gather(x_hbm, lut_hbm, out_hbm, idx_v, rows, gsem, osem):
        wid = lax.axis_index("s") * NUM_CORES + lax.axis_index("c")
        base = wid * _B_PER_W
        pltpu.sync_copy(x_hbm.at[wid], idx_v)

        def start_gather(t, b):
            pltpu.make_async_copy(
                lut_hbm.at[idx_v.at[t]], rows.at[b], gsem.at[b]
            ).start()

        def wait_gather(b):
            pltpu.make_async_copy(
                lut_hbm.at[idx_v.at[0]], rows.at[b], gsem.at[b]
            ).wait()

        def start_out(t, b):
            pltpu.make_async_copy(
                rows.at[b], out_hbm.at[pl.ds(base + 128 * t, 128)], osem.at[b]
            ).start()

        def wait_out(b):
            pltpu.make_async_copy(
                rows.at[b], out_hbm.at[pl.ds(base, 128)], osem.at[b]
            ).wait()

        def step(t, b):
            bg = (b + LEAD) % NBUF

            @pl.when(t + LEAD < _NBLK)
            def _():
                wait_out(bg)
                start_gather(t + LEAD, bg)

            wait_gather(b)
            start_out(t, b)

        for b in range(LEAD):
            start_gather(b, b)

        for b in range(NBUF):
            bg = (b + LEAD) % NBUF
            if b >= NBUF - LEAD:
                wait_out(bg)
            start_gather(b + LEAD, bg)
            wait_gather(b)
            start_out(b, b)

        def group(g, carry):
            for b in range(NBUF):
                step(g * NBUF + b, b)
            return carry

        lax.fori_loop(1, _NBLK // NBUF, group, 0)

        for b in range(NBUF):
            wait_out(b)

    return gather


def kernel(x, lut):
    lutT = jnp.transpose(lut)
    lut_pk = _repack_tc(lutT)
    x3 = x.astype(jnp.int32).reshape(NW, _NBLK, 128)
    out = _gather_kernel()(x3, lut_pk)
    return out[:, :D].reshape(BATCH, SEQ, D)


# speedup vs baseline: 4.2439x; 1.1133x over previous
import functools

import jax
import jax.numpy as jnp
from jax import lax
from jax.experimental import pallas as pl
from jax.experimental.pallas import tpu as pltpu
from jax.experimental.pallas import tpu_sc as plsc

VOCAB = 1000000
D = 64
BATCH = 4096
SEQ = 200

NUM_CORES = 2
NUM_SUBCORES = 16
NW = NUM_CORES * NUM_SUBCORES

TBLK = 32768
NBUF = 5
LEAD = 2

_B_PER_W = BATCH * SEQ // NW
_NBLK = _B_PER_W // 128


def _mesh():
    return plsc.VectorSubcoreMesh(core_axis_name="c", subcore_axis_name="s")


def _repack_tc(lutT):
    def body(x_ref, o_ref):
        o_ref[:, 0:D] = jnp.transpose(x_ref[...])

    grid = (VOCAB + TBLK - 1) // TBLK
    return pl.pallas_call(
        body,
        grid=(grid,),
        in_specs=[pl.BlockSpec((D, TBLK), lambda i: (0, i))],
        out_specs=pl.BlockSpec((TBLK, 2 * D), lambda i: (i, 0)),
        out_shape=jax.ShapeDtypeStruct((VOCAB, 2 * D), jnp.float32),
    )(lutT)


@functools.lru_cache(maxsize=None)
def _gather_kernel():
    @functools.partial(
        pl.kernel,
        mesh=_mesh(),
        out_type=jax.ShapeDtypeStruct((BATCH * SEQ, 2 * D), jnp.float32),
        compiler_params=pltpu.CompilerParams(
            use_tc_tiling_on_sc=False, needs_layout_passes=False
        ),
        scratch_types=[
            pltpu.VMEM((_NBLK, 128), jnp.int32),
            pltpu.VMEM((NBUF, 128, 2 * D), jnp.float32),
            pltpu.SemaphoreType.DMA((NBUF,)),
            pltpu.SemaphoreType.DMA((NBUF,)),
        ],
    )
    def gather(x_hbm, lut_hbm, out_hbm, idx_v, rows, gsem, osem):
        wid = lax.axis_index("s") * NUM_CORES + lax.axis_index("c")
        base = wid * _B_PER_W
        pltpu.sync_copy(x_hbm.at[wid], idx_v)

        def start_gather(t, b):
            pltpu.make_async_copy(
                lut_hbm.at[idx_v.at[t]], rows.at[b], gsem.at[b]
            ).start()

        def wait_gather(b):
            pltpu.make_async_copy(
                lut_hbm.at[idx_v.at[0]], rows.at[b], gsem.at[b]
            ).wait()

        def start_out(t, b):
            pltpu.make_async_copy(
                rows.at[b, :, pl.ds(0, D)],
                out_hbm.at[pl.ds(base + 128 * t, 128), pl.ds(0, D)],
                osem.at[b]
            ).start()

        def wait_out(b):
            pltpu.make_async_copy(
                rows.at[b, :, pl.ds(0, D)],
                out_hbm.at[pl.ds(base, 128), pl.ds(0, D)], osem.at[b]
            ).wait()

        def step(t, b):
            bg = (b + LEAD) % NBUF

            @pl.when(t + LEAD < _NBLK)
            def _():
                wait_out(bg)
                start_gather(t + LEAD, bg)

            wait_gather(b)
            start_out(t, b)

        for b in range(LEAD):
            start_gather(b, b)

        for b in range(NBUF):
            bg = (b + LEAD) % NBUF
            if b >= NBUF - LEAD:
                wait_out(bg)
            start_gather(b + LEAD, bg)
            wait_gather(b)
            start_out(b, b)

        def group(g, carry):
            for b in range(NBUF):
                step(g * NBUF + b, b)
            return carry

        lax.fori_loop(1, _NBLK // NBUF, group, 0)

        for b in range(NBUF):
            wait_out(b)

    return gather


def kernel(x, lut):
    lutT = jnp.transpose(lut)
    lut_pk = _repack_tc(lutT)
    x3 = x.astype(jnp.int32).reshape(NW, _NBLK, 128)
    out = _gather_kernel()(x3, lut_pk)
    return out[:, :D].reshape(BATCH, SEQ, D)
